# baseline (speedup 1.0000x reference)
import functools

import jax
import jax.numpy as jnp
from jax import lax
from jax.experimental import pallas as pl
from jax.experimental.pallas import tpu as pltpu
from jax.experimental.pallas import tpu_sc as plsc

D = 128
N = D * D * D
TIME_STEP = 5

NC, NS, L = 2, 16, 16
NW = NC * NS
PER_W = N // NW
C = 1024
SLICES = 3 * C // 128
GROUPS = C // L
N_CHUNKS = PER_W // C

_F32 = jnp.float32
_I32 = jnp.int32


def _splat_i(v):
    return jnp.full((L,), v, _I32)


def _splat_f(v):
    return jnp.full((L,), v, _F32)


def _take(x, idx):
    return jnp.take_along_axis(x, idx, axis=0, mode="promise_in_bounds")


def _make_step(scale: float):
    mesh = plsc.VectorSubcoreMesh(
        core_axis_name="c", subcore_axis_name="s", num_cores=NC, num_subcores=NS
    )

    cpos = scale * (D - 1) / 2.0

    @functools.partial(
        pl.kernel,
        out_type=jax.ShapeDtypeStruct((3 * N,), _F32),
        mesh=mesh,
        scratch_types=[
            pltpu.VMEM((3 * C,), _F32),
            pltpu.VMEM((3 * C,), _F32),
            pltpu.VMEM((8 * 3 * C,), _F32),
            pltpu.VMEM((8, SLICES, 128), _I32),
            pltpu.VMEM((8 * C,), _F32),
            pltpu.SemaphoreType.DMA,
        ],
    )
    def step(tin, tout, inb, outb, gb, idxbuf, wbuf, sem):
        wid = lax.axis_index("s") * NC + lax.axis_index("c")
        lane = lax.iota(_I32, L)
        vm = []
        rm = []
        for m in range(3):
            s_m = lane + _splat_i(16 * m)
            v_m = lax.shift_right_logical(s_m * _splat_i(683), _splat_i(11))
            vm.append(v_m)
            rm.append(s_m - v_m * _splat_i(3))
        def chunk_body(j, _):
            rowbase = wid * PER_W + j * C

            pltpu.sync_copy(tin.at[pl.ds(3 * rowbase, 3 * C)], inb)

            def wgt_body(g, _):
                voxbase = g * L
                p = _splat_i(0) + voxbase + lane + rowbase

                i0 = lax.shift_right_logical(p, 14)
                i1 = lax.bitwise_and(lax.shift_right_logical(p, 7), _splat_i(127))
                i2 = lax.bitwise_and(p, _splat_i(127))

                fb = 3 * voxbase
                v0 = inb[pl.ds(fb, L)]
                v1 = inb[pl.ds(fb + L, L)]
                v2 = inb[pl.ds(fb + 2 * L, L)]

                def chan(c):
                    slot = lane * _splat_i(3) + _splat_i(c)
                    idx0 = slot
                    idx1 = slot - _splat_i(16)
                    idx2 = slot - _splat_i(32)
                    ge16 = slot >= _splat_i(16)
                    ge32 = slot >= _splat_i(32)
                    lo = _take(v0, jnp.where(ge16, _splat_i(0), idx0))
                    mid = _take(v1, jnp.where(ge16, jnp.maximum(idx1, _splat_i(0)),
                                              _splat_i(0)))
                    hi2 = _take(v2, jnp.where(ge32, idx2, _splat_i(0)))
                    return jnp.where(ge32, hi2, jnp.where(ge16, mid, lo))

                f0 = chan(0)
                f1 = chan(1)
                f2 = chan(2)

                zero = _splat_f(0.0)
                hi = _splat_f(float(D - 1))

                def axis_terms(i_int, f):
                    pos = i_int.astype(_F32) + f * cpos
                    pos = jnp.minimum(jnp.maximum(pos, zero), hi)
                    b = jnp.minimum(pos.astype(_I32), _splat_i(D - 2))
                    t = pos - b.astype(_F32)
                    return b, t

                b0, t0v = axis_terms(i0, f0)
                b1, t1v = axis_terms(i1, f1)
                b2, t2v = axis_terms(i2, f2)

                one = _splat_f(1.0)
                u0, u1, u2 = one - t0v, one - t1v, one - t2v

                base3 = (
                    lax.shift_left(b0, _splat_i(14))
                    + lax.shift_left(b1, _splat_i(7))
                    + b2
                ) * _splat_i(3)

                a00 = u0 * u1
                a01 = u0 * t1v
                a10 = t0v * u1
                a11 = t0v * t1v

                wts = (a00 * u2, a00 * t2v, a01 * u2, a01 * t2v,
                       a10 * u2, a10 * t2v, a11 * u2, a11 * t2v)
                for k in range(8):
                    wbuf[pl.ds(k * C + voxbase, L)] = wts[k]

                offs3 = (0, 3, 3 * 128, 3 * 129,
                         3 * 16384, 3 * 16385, 3 * 16512, 3 * 16513)
                for k in range(8):
                    for m in range(3):
                        idxv = _take(base3, vm[m]) + rm[m] + _splat_i(offs3[k])
                        fpos = 3 * voxbase + 16 * m
                        srow = lax.shift_right_logical(fpos, 7)
                        scol = lax.rem(fpos, jnp.int32(128))
                        idxbuf[k, srow, pl.ds(scol, L)] = idxv
                return 0

            lax.fori_loop(0, GROUPS, wgt_body, 0)

            copies = []
            for k in range(8):
                for s in range(SLICES):
                    copies.append(pltpu.async_copy(
                        tin.at[idxbuf.at[k, s]],
                        gb.at[pl.ds(k * 3 * C + s * 128, 128)],
                        sem,
                    ))
            for cp in copies:
                cp.wait()

            def mac_body(g, _):
                vb = g * L
                fb = 3 * vb
                ws = [wbuf[pl.ds(k * C + vb, L)] for k in range(8)]
                for m in range(3):
                    acc = inb[pl.ds(fb + m * L, L)]
                    for k in range(8):
                        wexp = _take(ws[k], vm[m])
                        gv = gb[pl.ds(k * 3 * C + fb + m * L, L)]
                        acc = acc + gv * wexp
                    if scale != 1.0:
                        acc = acc * _splat_f(scale)
                    outb[pl.ds(fb + m * L, L)] = acc
                return 0

            lax.fori_loop(0, GROUPS, mac_body, 0)

            pltpu.sync_copy(outb, tout.at[pl.ds(3 * rowbase, 3 * C)])
            return 0

        lax.fori_loop(0, N_CHUNKS, chunk_body, 0)

    return step


def kernel(velocity):
    tab = velocity.reshape(3 * N)
    step1 = _make_step(1.0 / (2.0 ** TIME_STEP))
    stepn = _make_step(1.0)
    tab = step1(tab)
    for _ in range(TIME_STEP - 1):
        tab = stepn(tab)
    return tab.reshape(N, 3)


# speedup vs baseline: 1.5634x; 1.5634x over previous
import functools

import jax
import jax.numpy as jnp
from jax import lax
from jax.experimental import pallas as pl
from jax.experimental.pallas import tpu as pltpu
from jax.experimental.pallas import tpu_sc as plsc

D = 128
N = D * D * D
TIME_STEP = 5

NC, NS, L = 2, 16, 16
NW = NC * NS
PER_W = N // NW
C = 1024
SLICES = C // 128
GROUPS = C // L
N_CHUNKS = PER_W // C
N_PAIRS = N_CHUNKS // 2

_F32 = jnp.float32
_I32 = jnp.int32


def _splat_i(v):
    return jnp.full((L,), v, _I32)


def _splat_f(v):
    return jnp.full((L,), v, _F32)


def _make_step(scale: float):
    mesh = plsc.VectorSubcoreMesh(
        core_axis_name="c", subcore_axis_name="s", num_cores=NC, num_subcores=NS
    )

    cpos = scale * (D - 1) / 2.0

    @functools.partial(
        pl.kernel,
        out_type=tuple(jax.ShapeDtypeStruct((N,), _F32) for _ in range(3)),
        mesh=mesh,
        scratch_types=[
            tuple(tuple(pltpu.VMEM((C,), _F32) for _ in range(3))
                  for _ in range(2)),
            tuple(tuple(pltpu.VMEM((C,), _F32) for _ in range(3))
                  for _ in range(2)),
            tuple(tuple(pltpu.VMEM((8 * C,), _F32) for _ in range(3))
                  for _ in range(2)),
            tuple(pltpu.VMEM((8, SLICES, 128), _I32) for _ in range(2)),
            tuple(pltpu.VMEM((8 * C,), _F32) for _ in range(2)),
            tuple(pltpu.SemaphoreType.DMA for _ in range(2)),
        ],
    )
    def step(t0, t1, t2, o0, o1, o2, inb, outb, gb, idxbuf, wbuf, sems):
        tins = (t0, t1, t2)
        touts = (o0, o1, o2)
        wid = lax.axis_index("s") * NC + lax.axis_index("c")
        lane = lax.iota(_I32, L)
        wbase = wid * PER_W

        def prepare(rowbase, par):
            for c in range(3):
                pltpu.sync_copy(tins[c].at[pl.ds(rowbase, C)], inb[par][c])

            def wgt_body(g, _):
                voxbase = g * L
                p = _splat_i(0) + voxbase + lane + rowbase

                i0 = lax.shift_right_logical(p, 14)
                i1 = lax.bitwise_and(lax.shift_right_logical(p, 7), _splat_i(127))
                i2 = lax.bitwise_and(p, _splat_i(127))

                f0 = inb[par][0][pl.ds(voxbase, L)]
                f1 = inb[par][1][pl.ds(voxbase, L)]
                f2 = inb[par][2][pl.ds(voxbase, L)]

                zero = _splat_f(0.0)
                hi = _splat_f(float(D - 1))

                def axis_terms(i_int, f):
                    pos = i_int.astype(_F32) + f * cpos
                    pos = jnp.minimum(jnp.maximum(pos, zero), hi)
                    b = jnp.minimum(pos.astype(_I32), _splat_i(D - 2))
                    t = pos - b.astype(_F32)
                    return b, t

                b0, t0v = axis_terms(i0, f0)
                b1, t1v = axis_terms(i1, f1)
                b2, t2v = axis_terms(i2, f2)

                one = _splat_f(1.0)
                u0, u1, u2 = one - t0v, one - t1v, one - t2v

                base_idx = (
                    lax.shift_left(b0, _splat_i(14))
                    + lax.shift_left(b1, _splat_i(7))
                    + b2
                )

                a00 = u0 * u1
                a01 = u0 * t1v
                a10 = t0v * u1
                a11 = t0v * t1v

                srow = lax.shift_right_logical(voxbase, 7)
                scol = lax.rem(voxbase, jnp.int32(128))

                offs = (0, 1, 128, 129, 16384, 16385, 16512, 16513)
                wts = (a00 * u2, a00 * t2v, a01 * u2, a01 * t2v,
                       a10 * u2, a10 * t2v, a11 * u2, a11 * t2v)
                for k in range(8):
                    idxbuf[par][k, srow, pl.ds(scol, L)] = (
                        base_idx + _splat_i(offs[k]))
                    wbuf[par][pl.ds(k * C + voxbase, L)] = wts[k]
                return 0

            lax.fori_loop(0, GROUPS, wgt_body, 0)

        def fire(par):
            for k in range(8):
                for s in range(SLICES):
                    for c in range(3):
                        pltpu.async_copy(
                            tins[c].at[idxbuf[par].at[k, s]],
                            gb[par][c].at[pl.ds(k * C + s * 128, 128)],
                            sems[par],
                        )

        def drain(par):
            for k in range(8):
                for s in range(SLICES):
                    for c in range(3):
                        pltpu.make_async_copy(
                            tins[c].at[idxbuf[par].at[k, s]],
                            gb[par][c].at[pl.ds(k * C + s * 128, 128)],
                            sems[par],
                        ).wait()

        def mac_and_out(rowbase, par):
            def mac_body(g, _):
                vb = g * L
                ws = [wbuf[par][pl.ds(k * C + vb, L)] for k in range(8)]
                for c in range(3):
                    acc = inb[par][c][pl.ds(vb, L)]
                    for k in range(8):
                        acc = acc + gb[par][c][pl.ds(k * C + vb, L)] * ws[k]
                    if scale != 1.0:
                        acc = acc * _splat_f(scale)
                    outb[par][c][pl.ds(vb, L)] = acc
                return 0

            lax.fori_loop(0, GROUPS, mac_body, 0)
            for c in range(3):
                pltpu.sync_copy(outb[par][c], touts[c].at[pl.ds(rowbase, C)])

        prepare(wbase, 0)
        fire(0)

        def pair_body(j2, _):
            even = wbase + (2 * j2) * C
            odd = even + C

            prepare(odd, 1)
            fire(1)

            drain(0)
            mac_and_out(even, 0)

            @pl.when(j2 < N_PAIRS - 1)
            def _prefetch():
                prepare(odd + C, 0)
                fire(0)

            drain(1)
            mac_and_out(odd, 1)
            return 0

        lax.fori_loop(0, N_PAIRS, pair_body, 0)

    return step


def kernel(velocity):
    t0, t1, t2 = (velocity[:, c] for c in range(3))
    step1 = _make_step(1.0 / (2.0 ** TIME_STEP))
    stepn = _make_step(1.0)
    t0, t1, t2 = step1(t0, t1, t2)
    for _ in range(TIME_STEP - 1):
        t0, t1, t2 = stepn(t0, t1, t2)
    return jnp.stack([t0, t1, t2], axis=1)


# speedup vs baseline: 1.5700x; 1.0042x over previous
import functools

import jax
import jax.numpy as jnp
from jax import lax
from jax.experimental import pallas as pl
from jax.experimental.pallas import tpu as pltpu
from jax.experimental.pallas import tpu_sc as plsc

D = 128
N = D * D * D
TIME_STEP = 5

NC, NS, L = 2, 16, 16
NW = NC * NS
PER_W = N // NW
C = 1024
IDXW = C
GROUPS = C // L
N_CHUNKS = PER_W // C
N_PAIRS = N_CHUNKS // 2

_F32 = jnp.float32
_I32 = jnp.int32


def _splat_i(v):
    return jnp.full((L,), v, _I32)


def _splat_f(v):
    return jnp.full((L,), v, _F32)


def _make_step(scale: float):
    mesh = plsc.VectorSubcoreMesh(
        core_axis_name="c", subcore_axis_name="s", num_cores=NC, num_subcores=NS
    )

    cpos = scale * (D - 1) / 2.0

    @functools.partial(
        pl.kernel,
        out_type=tuple(jax.ShapeDtypeStruct((N,), _F32) for _ in range(3)),
        mesh=mesh,
        scratch_types=[
            tuple(tuple(pltpu.VMEM((C,), _F32) for _ in range(3))
                  for _ in range(2)),
            tuple(tuple(pltpu.VMEM((C,), _F32) for _ in range(3))
                  for _ in range(2)),
            tuple(tuple(pltpu.VMEM((8 * C,), _F32) for _ in range(3))
                  for _ in range(2)),
            tuple(pltpu.VMEM((8, 1, IDXW), _I32) for _ in range(2)),
            tuple(pltpu.VMEM((8 * C,), _F32) for _ in range(2)),
            tuple(pltpu.SemaphoreType.DMA for _ in range(2)),
        ],
    )
    def step(t0, t1, t2, o0, o1, o2, inb, outb, gb, idxbuf, wbuf, sems):
        tins = (t0, t1, t2)
        touts = (o0, o1, o2)
        wid = lax.axis_index("s") * NC + lax.axis_index("c")
        lane = lax.iota(_I32, L)
        wbase = wid * PER_W

        def prepare(rowbase, par):
            for c in range(3):
                pltpu.sync_copy(tins[c].at[pl.ds(rowbase, C)], inb[par][c])

            def wgt_body(g, _):
                voxbase = g * L
                p = _splat_i(0) + voxbase + lane + rowbase

                i0 = lax.shift_right_logical(p, 14)
                i1 = lax.bitwise_and(lax.shift_right_logical(p, 7), _splat_i(127))
                i2 = lax.bitwise_and(p, _splat_i(127))

                f0 = inb[par][0][pl.ds(voxbase, L)]
                f1 = inb[par][1][pl.ds(voxbase, L)]
                f2 = inb[par][2][pl.ds(voxbase, L)]

                zero = _splat_f(0.0)
                hi = _splat_f(float(D - 1))

                def axis_terms(i_int, f):
                    pos = i_int.astype(_F32) + f * cpos
                    pos = jnp.minimum(jnp.maximum(pos, zero), hi)
                    b = jnp.minimum(pos.astype(_I32), _splat_i(D - 2))
                    t = pos - b.astype(_F32)
                    return b, t

                b0, t0v = axis_terms(i0, f0)
                b1, t1v = axis_terms(i1, f1)
                b2, t2v = axis_terms(i2, f2)

                one = _splat_f(1.0)
                u0, u1, u2 = one - t0v, one - t1v, one - t2v

                base_idx = (
                    lax.shift_left(b0, _splat_i(14))
                    + lax.shift_left(b1, _splat_i(7))
                    + b2
                )

                a00 = u0 * u1
                a01 = u0 * t1v
                a10 = t0v * u1
                a11 = t0v * t1v


                offs = (0, 1, 128, 129, 16384, 16385, 16512, 16513)
                wts = (a00 * u2, a00 * t2v, a01 * u2, a01 * t2v,
                       a10 * u2, a10 * t2v, a11 * u2, a11 * t2v)
                for k in range(8):
                    idxbuf[par][k, 0, pl.ds(voxbase, L)] = (
                        base_idx + _splat_i(offs[k]))
                    wbuf[par][pl.ds(k * C + voxbase, L)] = wts[k]
                return 0

            lax.fori_loop(0, GROUPS, wgt_body, 0)

        def fire(par):
            for k in range(8):
                for c in range(3):
                    pltpu.async_copy(
                        tins[c].at[idxbuf[par].at[k, 0]],
                        gb[par][c].at[pl.ds(k * C, C)],
                        sems[par],
                    )

        def drain(par):
            for c in range(3):
                pltpu.make_async_copy(
                    tins[c].at[pl.ds(0, 8 * C)],
                    gb[par][c],
                    sems[par],
                ).wait()

        def mac_and_out(rowbase, par):
            def mac_body(g, _):
                vb = g * L
                ws = [wbuf[par][pl.ds(k * C + vb, L)] for k in range(8)]
                for c in range(3):
                    acc = inb[par][c][pl.ds(vb, L)]
                    for k in range(8):
                        acc = acc + gb[par][c][pl.ds(k * C + vb, L)] * ws[k]
                    if scale != 1.0:
                        acc = acc * _splat_f(scale)
                    outb[par][c][pl.ds(vb, L)] = acc
                return 0

            lax.fori_loop(0, GROUPS, mac_body, 0)
            for c in range(3):
                pltpu.sync_copy(outb[par][c], touts[c].at[pl.ds(rowbase, C)])

        prepare(wbase, 0)
        fire(0)

        def pair_body(j2, _):
            even = wbase + (2 * j2) * C
            odd = even + C

            prepare(odd, 1)
            fire(1)

            drain(0)
            mac_and_out(even, 0)

            @pl.when(j2 < N_PAIRS - 1)
            def _prefetch():
                prepare(odd + C, 0)
                fire(0)

            drain(1)
            mac_and_out(odd, 1)
            return 0

        lax.fori_loop(0, N_PAIRS, pair_body, 0)

    return step


def kernel(velocity):
    t0, t1, t2 = (velocity[:, c] for c in range(3))
    step1 = _make_step(1.0 / (2.0 ** TIME_STEP))
    stepn = _make_step(1.0)
    t0, t1, t2 = step1(t0, t1, t2)
    for _ in range(TIME_STEP - 1):
        t0, t1, t2 = stepn(t0, t1, t2)
    return jnp.stack([t0, t1, t2], axis=1)


# speedup vs baseline: 2.7155x; 1.7296x over previous
import functools

import jax
import jax.numpy as jnp
from jax import lax
from jax.experimental import pallas as pl
from jax.experimental.pallas import tpu as pltpu
from jax.experimental.pallas import tpu_sc as plsc
from jax.sharding import PartitionSpec as P

D = 128
N = D * D * D
TIME_STEP = 5

NC, NS, L = 2, 16, 16
NW = NC * NS
C = 1024
GROUPS = C // L

_F32 = jnp.float32
_I32 = jnp.int32


def _splat_i(v):
    return jnp.full((L,), v, _I32)


def _splat_f(v):
    return jnp.full((L,), v, _F32)


def _make_step(scale: float, ndev: int):
    mesh = plsc.VectorSubcoreMesh(
        core_axis_name="c", subcore_axis_name="s", num_cores=NC, num_subcores=NS
    )

    n_shard = N // ndev
    per_w = n_shard // NW
    n_chunks = per_w // C
    n_pairs = n_chunks // 2

    cpos = scale * (D - 1) / 2.0

    @functools.partial(
        pl.kernel,
        out_type=tuple(jax.ShapeDtypeStruct((n_shard,), _F32) for _ in range(3)),
        mesh=mesh,
        scratch_types=[
            pltpu.VMEM((L,), _I32),
            tuple(tuple(pltpu.VMEM((C,), _F32) for _ in range(3))
                  for _ in range(2)),
            tuple(tuple(pltpu.VMEM((C,), _F32) for _ in range(3))
                  for _ in range(2)),
            tuple(tuple(pltpu.VMEM((8 * C,), _F32) for _ in range(3))
                  for _ in range(2)),
            tuple(pltpu.VMEM((8, 1, C), _I32) for _ in range(2)),
            tuple(pltpu.VMEM((8 * C,), _F32) for _ in range(2)),
            tuple(pltpu.SemaphoreType.DMA for _ in range(2)),
        ],
    )
    def step(off16, t0, t1, t2, m0, m1, m2, o0, o1, o2,
             offv, inb, outb, gb, idxbuf, wbuf, sems):
        tins = (t0, t1, t2)
        mins = (m0, m1, m2)
        touts = (o0, o1, o2)
        wid = lax.axis_index("s") * NC + lax.axis_index("c")
        lane = lax.iota(_I32, L)
        wbase = wid * per_w

        pltpu.sync_copy(off16, offv)
        dev_off = offv[...]

        def prepare(rowbase, par):
            for c in range(3):
                pltpu.sync_copy(mins[c].at[pl.ds(rowbase, C)], inb[par][c])

            def wgt_body(g, _):
                voxbase = g * L
                p = _splat_i(0) + voxbase + lane + rowbase + dev_off

                i0 = lax.shift_right_logical(p, 14)
                i1 = lax.bitwise_and(lax.shift_right_logical(p, 7), _splat_i(127))
                i2 = lax.bitwise_and(p, _splat_i(127))

                f0 = inb[par][0][pl.ds(voxbase, L)]
                f1 = inb[par][1][pl.ds(voxbase, L)]
                f2 = inb[par][2][pl.ds(voxbase, L)]

                zero = _splat_f(0.0)
                hi = _splat_f(float(D - 1))

                def axis_terms(i_int, f):
                    pos = i_int.astype(_F32) + f * cpos
                    pos = jnp.minimum(jnp.maximum(pos, zero), hi)
                    b = jnp.minimum(pos.astype(_I32), _splat_i(D - 2))
                    t = pos - b.astype(_F32)
                    return b, t

                b0, t0v = axis_terms(i0, f0)
                b1, t1v = axis_terms(i1, f1)
                b2, t2v = axis_terms(i2, f2)

                one = _splat_f(1.0)
                u0, u1, u2 = one - t0v, one - t1v, one - t2v

                base_idx = (
                    lax.shift_left(b0, _splat_i(14))
                    + lax.shift_left(b1, _splat_i(7))
                    + b2
                )

                a00 = u0 * u1
                a01 = u0 * t1v
                a10 = t0v * u1
                a11 = t0v * t1v

                offs = (0, 1, 128, 129, 16384, 16385, 16512, 16513)
                wts = (a00 * u2, a00 * t2v, a01 * u2, a01 * t2v,
                       a10 * u2, a10 * t2v, a11 * u2, a11 * t2v)
                for k in range(8):
                    idxbuf[par][k, 0, pl.ds(voxbase, L)] = (
                        base_idx + _splat_i(offs[k]))
                    wbuf[par][pl.ds(k * C + voxbase, L)] = wts[k]
                return 0

            lax.fori_loop(0, GROUPS, wgt_body, 0)

        def fire(par):
            for k in range(8):
                for c in range(3):
                    pltpu.async_copy(
                        tins[c].at[idxbuf[par].at[k, 0]],
                        gb[par][c].at[pl.ds(k * C, C)],
                        sems[par],
                    )

        def drain(par):
            for c in range(3):
                pltpu.make_async_copy(
                    tins[c].at[pl.ds(0, 8 * C)],
                    gb[par][c],
                    sems[par],
                ).wait()

        def mac_and_out(rowbase, par):
            def mac_body(g, _):
                vb = g * L
                ws = [wbuf[par][pl.ds(k * C + vb, L)] for k in range(8)]
                for c in range(3):
                    acc = inb[par][c][pl.ds(vb, L)]
                    for k in range(8):
                        acc = acc + gb[par][c][pl.ds(k * C + vb, L)] * ws[k]
                    if scale != 1.0:
                        acc = acc * _splat_f(scale)
                    outb[par][c][pl.ds(vb, L)] = acc
                return 0

            lax.fori_loop(0, GROUPS, mac_body, 0)
            for c in range(3):
                pltpu.sync_copy(outb[par][c], touts[c].at[pl.ds(rowbase, C)])

        prepare(wbase, 0)
        fire(0)

        def pair_body(j2, _):
            even = wbase + (2 * j2) * C
            odd = even + C

            prepare(odd, 1)
            fire(1)

            drain(0)
            mac_and_out(even, 0)

            @pl.when(j2 < n_pairs - 1)
            def _prefetch():
                prepare(odd + C, 0)
                fire(0)

            drain(1)
            mac_and_out(odd, 1)
            return 0

        lax.fori_loop(0, n_pairs, pair_body, 0)

    return step


def kernel(velocity):
    devs = jax.devices()
    ndev = 2 if len(devs) >= 2 else 1
    n_shard = N // ndev

    step1 = _make_step(1.0 / (2.0 ** TIME_STEP), ndev)
    stepn = _make_step(1.0, ndev)

    if ndev == 1:
        off16 = jnp.zeros((L,), _I32)
        my = [velocity[:, c] for c in range(3)]
        for s in range(TIME_STEP):
            stp = step1 if s == 0 else stepn
            my = list(stp(off16, *my, *my))
        return jnp.stack(my, axis=1)

    mesh = jax.sharding.Mesh(devs[:ndev], ("d",))

    def body(vel):
        d = lax.axis_index("d")
        off16 = jnp.full((L,), d * n_shard, _I32)
        full = [vel[:, c] for c in range(3)]
        my = [lax.dynamic_slice_in_dim(full[c], d * n_shard, n_shard)
              for c in range(3)]
        for s in range(TIME_STEP):
            stp = step1 if s == 0 else stepn
            my = list(stp(off16, *full, *my))
            if s != TIME_STEP - 1:
                full = [lax.all_gather(m, "d", tiled=True) for m in my]
        return jnp.stack(my, axis=1)

    return jax.shard_map(
        body, mesh=mesh, in_specs=(P(None, None),), out_specs=P("d", None)
    )(velocity)
